# baseline (device time: 302180 ns/iter reference)
import jax
import jax.numpy as jnp
import numpy as np
from jax import lax
from jax.experimental import pallas as pl
from jax.experimental.pallas import tpu as pltpu

N = 32
B, S, C = 4, 1024, 512
OUT_N = 512
ROWS = B * S
CHUNK = ROWS // N
TAPS = 4


def _build_ring_tables():
    ident = (list(range(N)),
             [(p + 1) % N for p in range(N)],
             [(p - 1) % N for p in range(N)])
    try:
        import distributed_mesh_v7x as dm
        mesh = dm.get_mesh("i", world_size=N)
        devs = list(mesh.devices.flat)
        coords = [tuple(d.coords) for d in devs]
        if len(set(coords)) != N or any(len(c) != 3 for c in coords):
            return ident
        axes = [sorted({c[i] for c in coords}) for i in range(3)]
        sizes = [len(a) for a in axes]
        if sorted(sizes) != [2, 4, 4]:
            return ident
        a2 = sizes.index(2)
        a4 = [i for i in range(3) if i != a2]

        def mk(v2, u, v):
            c = [0, 0, 0]
            c[a2], c[a4[0]], c[a4[1]] = v2, u, v
            return tuple(c)

        us, vs = axes[a4[0]], axes[a4[1]]
        lo, hi = axes[a2]
        cycle = []
        for vi, v in enumerate(vs):
            row = us if vi % 2 == 0 else us[::-1]
            cycle += [mk(lo, u, v) for u in row]
        for vi, v in enumerate(reversed(vs)):
            row = us if vi % 2 == 0 else us[::-1]
            cycle += [mk(hi, u, v) for u in row]
        if set(cycle) != set(coords):
            return ident
        for i in range(N):
            d = sum(abs(a - b) for a, b in zip(cycle[i], cycle[(i + 1) % N]))
            if d != 1:
                return ident
        pos_of = {c: i for i, c in enumerate(cycle)}
        ringpos = [pos_of[c] for c in coords]
        order = [0] * N
        for p, rp in enumerate(ringpos):
            order[rp] = p
        right = [order[(ringpos[p] + 1) % N] for p in range(N)]
        left = [order[(ringpos[p] - 1) % N] for p in range(N)]
        return ringpos, right, left
    except Exception:
        return ident


_RINGPOS, _RIGHT, _LEFT = _build_ring_tables()


def _body(scal_ref, x_ref, k_ref, wp_ref, out_ref,
          pad_ref, part_ref, rsbuf_ref,
          rs_send, rs_recv, ag_send, ag_recv):
    rp = scal_ref[0]
    rt = scal_ref[1]
    lt = scal_ref[2]

    barrier_sem = pltpu.get_barrier_semaphore()
    pl.semaphore_signal(barrier_sem, inc=1, device_id=(lt,),
                        device_id_type=pl.DeviceIdType.MESH)
    pl.semaphore_signal(barrier_sem, inc=1, device_id=(rt,),
                        device_id_type=pl.DeviceIdType.MESH)

    kv = k_ref[:, :]
    wpv = wp_ref[:, :]
    for b in range(B):
        pad_ref[b, 0:TAPS - 1, :] = jnp.zeros((TAPS - 1, C), jnp.float32)
        pad_ref[b, TAPS - 1:TAPS - 1 + S, :] = x_ref[b]
    for b in range(B):
        acc = pad_ref[b, 0:S, :] * kv[0:1, :]
        for t in range(1, TAPS):
            acc = acc + pad_ref[b, t:t + S, :] * kv[t:t + 1, :]
        a = acc * (1.0 / (1.0 + jnp.exp(-acc)))
        part_ref[pl.ds(b * S, S), :] = jnp.dot(
            a, wpv, preferred_element_type=jnp.float32)

    pl.semaphore_wait(barrier_sem, 2)

    for s in range(N - 1):
        send_idx = (rp - s) % N
        recv_idx = (rp - s - 1) % N
        rdma = pltpu.make_async_remote_copy(
            src_ref=part_ref.at[pl.ds(send_idx * CHUNK, CHUNK), :],
            dst_ref=rsbuf_ref.at[s],
            send_sem=rs_send.at[s],
            recv_sem=rs_recv.at[s],
            device_id=rt,
            device_id_type=pl.DeviceIdType.LOGICAL,
        )
        rdma.start()
        rdma.wait()
        part_ref[pl.ds(recv_idx * CHUNK, CHUNK), :] = (
            part_ref[pl.ds(recv_idx * CHUNK, CHUNK), :] + rsbuf_ref[s]
        )

    own = (rp + 1) % N
    out_ref[pl.ds(own * CHUNK, CHUNK), :] = part_ref[pl.ds(own * CHUNK, CHUNK), :]

    for s in range(N - 1):
        send_idx = (own - s) % N
        rdma = pltpu.make_async_remote_copy(
            src_ref=out_ref.at[pl.ds(send_idx * CHUNK, CHUNK), :],
            dst_ref=out_ref.at[pl.ds(send_idx * CHUNK, CHUNK), :],
            send_sem=ag_send.at[s],
            recv_sem=ag_recv.at[s],
            device_id=rt,
            device_id_type=pl.DeviceIdType.LOGICAL,
        )
        rdma.start()
        rdma.wait()


def kernel(x, k, Wp):
    my = lax.axis_index("i")
    rp = jnp.asarray(_RINGPOS, jnp.int32)[my]
    rt = jnp.asarray(_RIGHT, jnp.int32)[my]
    lt = jnp.asarray(_LEFT, jnp.int32)[my]
    scalars = jnp.stack([rp, rt, lt]).astype(jnp.int32)

    out = pl.pallas_call(
        _body,
        out_shape=jax.ShapeDtypeStruct((ROWS, OUT_N), jnp.float32),
        in_specs=[
            pl.BlockSpec(memory_space=pltpu.SMEM),
            pl.BlockSpec(memory_space=pltpu.VMEM),
            pl.BlockSpec(memory_space=pltpu.VMEM),
            pl.BlockSpec(memory_space=pltpu.VMEM),
        ],
        out_specs=pl.BlockSpec(memory_space=pltpu.VMEM),
        scratch_shapes=[
            pltpu.VMEM((B, TAPS - 1 + S, C), jnp.float32),
            pltpu.VMEM((ROWS, OUT_N), jnp.float32),
            pltpu.VMEM((N - 1, CHUNK, OUT_N), jnp.float32),
            pltpu.SemaphoreType.DMA((N - 1,)),
            pltpu.SemaphoreType.DMA((N - 1,)),
            pltpu.SemaphoreType.DMA((N - 1,)),
            pltpu.SemaphoreType.DMA((N - 1,)),
        ],
        compiler_params=pltpu.CompilerParams(collective_id=0),
    )(scalars, x, k, Wp)
    return out.reshape(B, S, OUT_N)


# device time: 163101 ns/iter; 1.8527x vs baseline; 1.8527x over previous
import jax
import jax.numpy as jnp
import numpy as np
from jax import lax
from jax.experimental import pallas as pl
from jax.experimental.pallas import tpu as pltpu

N = 32
B, S, C = 4, 1024, 512
OUT_N = 512
ROWS = B * S
TAPS = 4

NA = 8
NB = 4
AC = ROWS // NA
H = AC // 2
BC = H // NB


def _build_tables():
    fb = ([p // NB for p in range(N)],
          [((p // NB + 1) % NA) * NB + p % NB for p in range(N)],
          [((p // NB - 1) % NA) * NB + p % NB for p in range(N)],
          [p % NB for p in range(N)],
          [(p // NB) * NB + (p + 1) % NB for p in range(N)],
          [(p // NB) * NB + (p - 1) % NB for p in range(N)])
    try:
        import distributed_mesh_v7x as dm
        mesh = dm.get_mesh("i", world_size=N)
        devs = list(mesh.devices.flat)
        coords = [tuple(d.coords) for d in devs]
        if len(set(coords)) != N or any(len(c) != 3 for c in coords):
            return fb
        axes = [sorted({c[i] for c in coords}) for i in range(3)]
        sizes = [len(a) for a in axes]
        if sorted(sizes) != [2, 4, 4]:
            return fb
        a2 = sizes.index(2)
        a4 = [i for i in range(3) if i != a2]
        us = axes[a4[0]]
        vs = axes[a4[1]]
        lo, hi = axes[a2]
        cyc = [(lo, u) for u in us] + [(hi, u) for u in reversed(us)]
        posA_of = {xu: i for i, xu in enumerate(cyc)}
        log_of = {c: p for p, c in enumerate(coords)}

        def at(c, i2, iu, iv):
            t = [0, 0, 0]
            t[a2], t[a4[0]], t[a4[1]] = i2, iu, iv
            return tuple(t)

        posA = [0] * N
        rtA = [0] * N
        ltA = [0] * N
        posB = [0] * N
        rtB = [0] * N
        ltB = [0] * N
        for p, c in enumerate(coords):
            i2, iu, iv = c[a2], c[a4[0]], c[a4[1]]
            pa = posA_of[(i2, iu)]
            pb = vs.index(iv)
            posA[p] = pa
            posB[p] = pb
            nxt = cyc[(pa + 1) % NA]
            prv = cyc[(pa - 1) % NA]
            rtA[p] = log_of[at(c, nxt[0], nxt[1], iv)]
            ltA[p] = log_of[at(c, prv[0], prv[1], iv)]
            rtB[p] = log_of[at(c, i2, iu, vs[(pb + 1) % NB])]
            ltB[p] = log_of[at(c, i2, iu, vs[(pb - 1) % NB])]
        return posA, rtA, ltA, posB, rtB, ltB
    except Exception:
        return fb


_POSA, _RTA, _LTA, _POSB, _RTB, _LTB = _build_tables()


def _body(scal_ref, x_ref, k_ref, wp_ref, out_ref,
          pad_ref, part_ref, bufAR, bufAL, bufBR, bufBL,
          rsA_sR, rsA_rR, rsA_sL, rsA_rL,
          rsB_sR, rsB_rR, rsB_sL, rsB_rL,
          agB_sR, agB_rR, agB_sL, agB_rL,
          agA_sR, agA_rR, agA_sL, agA_rL):
    posA = scal_ref[0]
    rtA = scal_ref[1]
    ltA = scal_ref[2]
    posB = scal_ref[3]
    rtB = scal_ref[4]
    ltB = scal_ref[5]

    barrier_sem = pltpu.get_barrier_semaphore()
    for nbr in (rtA, ltA, rtB, ltB):
        pl.semaphore_signal(barrier_sem, inc=1, device_id=(nbr,),
                            device_id_type=pl.DeviceIdType.MESH)

    kv = k_ref[:, :]
    wpv = wp_ref[:, :]
    for b in range(B):
        pad_ref[b, 0:TAPS - 1, :] = jnp.zeros((TAPS - 1, C), jnp.float32)
        pad_ref[b, TAPS - 1:TAPS - 1 + S, :] = x_ref[b]
    for b in range(B):
        acc = pad_ref[b, 0:S, :] * kv[0:1, :]
        for t in range(1, TAPS):
            acc = acc + pad_ref[b, t:t + S, :] * kv[t:t + 1, :]
        a = acc * (1.0 / (1.0 + jnp.exp(-acc)))
        part_ref[pl.ds(b * S, S), :] = jnp.dot(
            a, wpv, preferred_element_type=jnp.float32)

    pl.semaphore_wait(barrier_sem, 4)

    def copy(src, dst, ssem, rsem, dev):
        r = pltpu.make_async_remote_copy(
            src_ref=src, dst_ref=dst, send_sem=ssem, recv_sem=rsem,
            device_id=dev, device_id_type=pl.DeviceIdType.LOGICAL)
        r.start()
        return r

    for s in range(NA - 1):
        csR = ((posA - s) % NA) * AC
        crR = ((posA - s - 1) % NA) * AC
        csL = ((posA + s) % NA) * AC + H
        crL = ((posA + s + 1) % NA) * AC + H
        rR = copy(part_ref.at[pl.ds(csR, H), :], bufAR.at[s],
                  rsA_sR.at[s], rsA_rR.at[s], rtA)
        rL = copy(part_ref.at[pl.ds(csL, H), :], bufAL.at[s],
                  rsA_sL.at[s], rsA_rL.at[s], ltA)
        rR.wait_recv()
        part_ref[pl.ds(crR, H), :] = part_ref[pl.ds(crR, H), :] + bufAR[s]
        rL.wait_recv()
        part_ref[pl.ds(crL, H), :] = part_ref[pl.ds(crL, H), :] + bufAL[s]
        rR.wait_send()
        rL.wait_send()

    rbase = ((posA + 1) % NA) * AC
    lbase = ((posA - 1) % NA) * AC + H

    for s in range(NB - 1):
        sR = rbase + ((posB - s) % NB) * BC
        aR = rbase + ((posB - s - 1) % NB) * BC
        sL = lbase + ((posB + s) % NB) * BC
        aL = lbase + ((posB + s + 1) % NB) * BC
        rR = copy(part_ref.at[pl.ds(sR, BC), :], bufBR.at[s],
                  rsB_sR.at[s], rsB_rR.at[s], rtB)
        rL = copy(part_ref.at[pl.ds(sL, BC), :], bufBL.at[s],
                  rsB_sL.at[s], rsB_rL.at[s], ltB)
        rR.wait_recv()
        part_ref[pl.ds(aR, BC), :] = part_ref[pl.ds(aR, BC), :] + bufBR[s]
        rL.wait_recv()
        part_ref[pl.ds(aL, BC), :] = part_ref[pl.ds(aL, BC), :] + bufBL[s]
        rR.wait_send()
        rL.wait_send()

    ownR = rbase + ((posB + 1) % NB) * BC
    ownL = lbase + ((posB - 1) % NB) * BC
    out_ref[pl.ds(ownR, BC), :] = part_ref[pl.ds(ownR, BC), :]
    out_ref[pl.ds(ownL, BC), :] = part_ref[pl.ds(ownL, BC), :]

    for s in range(NB - 1):
        sR = rbase + ((posB + 1 - s) % NB) * BC
        sL = lbase + ((posB - 1 + s) % NB) * BC
        rR = copy(out_ref.at[pl.ds(sR, BC), :], out_ref.at[pl.ds(sR, BC), :],
                  agB_sR.at[s], agB_rR.at[s], rtB)
        rL = copy(out_ref.at[pl.ds(sL, BC), :], out_ref.at[pl.ds(sL, BC), :],
                  agB_sL.at[s], agB_rL.at[s], ltB)
        rR.wait_recv()
        rL.wait_recv()
        rR.wait_send()
        rL.wait_send()

    for s in range(NA - 1):
        sR = ((posA + 1 - s) % NA) * AC
        sL = ((posA - 1 + s) % NA) * AC + H
        rR = copy(out_ref.at[pl.ds(sR, H), :], out_ref.at[pl.ds(sR, H), :],
                  agA_sR.at[s], agA_rR.at[s], rtA)
        rL = copy(out_ref.at[pl.ds(sL, H), :], out_ref.at[pl.ds(sL, H), :],
                  agA_sL.at[s], agA_rL.at[s], ltA)
        rR.wait_recv()
        rL.wait_recv()
        rR.wait_send()
        rL.wait_send()


def kernel(x, k, Wp):
    my = lax.axis_index("i")
    scalars = jnp.stack([
        jnp.asarray(_POSA, jnp.int32)[my],
        jnp.asarray(_RTA, jnp.int32)[my],
        jnp.asarray(_LTA, jnp.int32)[my],
        jnp.asarray(_POSB, jnp.int32)[my],
        jnp.asarray(_RTB, jnp.int32)[my],
        jnp.asarray(_LTB, jnp.int32)[my],
    ]).astype(jnp.int32)

    sem7 = pltpu.SemaphoreType.DMA((NA - 1,))
    sem3 = pltpu.SemaphoreType.DMA((NB - 1,))
    out = pl.pallas_call(
        _body,
        out_shape=jax.ShapeDtypeStruct((ROWS, OUT_N), jnp.float32),
        in_specs=[
            pl.BlockSpec(memory_space=pltpu.SMEM),
            pl.BlockSpec(memory_space=pltpu.VMEM),
            pl.BlockSpec(memory_space=pltpu.VMEM),
            pl.BlockSpec(memory_space=pltpu.VMEM),
        ],
        out_specs=pl.BlockSpec(memory_space=pltpu.VMEM),
        scratch_shapes=[
            pltpu.VMEM((B, TAPS - 1 + S, C), jnp.float32),
            pltpu.VMEM((ROWS, OUT_N), jnp.float32),
            pltpu.VMEM((NA - 1, H, OUT_N), jnp.float32),
            pltpu.VMEM((NA - 1, H, OUT_N), jnp.float32),
            pltpu.VMEM((NB - 1, BC, OUT_N), jnp.float32),
            pltpu.VMEM((NB - 1, BC, OUT_N), jnp.float32),
            sem7, sem7, sem7, sem7,
            sem3, sem3, sem3, sem3,
            sem3, sem3, sem3, sem3,
            sem7, sem7, sem7, sem7,
        ],
        compiler_params=pltpu.CompilerParams(collective_id=0),
    )(scalars, x, k, Wp)
    return out.reshape(B, S, OUT_N)


# device time: 136877 ns/iter; 2.2077x vs baseline; 1.1916x over previous
import jax
import jax.numpy as jnp
import numpy as np
from jax import lax
from jax.experimental import pallas as pl
from jax.experimental.pallas import tpu as pltpu

N = 32
B, S, C = 4, 1024, 512
OUT_N = 512
ROWS = B * S
TAPS = 4

NA = 8
NB = 4
AC = ROWS // NA
H = AC // 2
BC = H // NB


def _build_tables():
    fb = ([p // NB for p in range(N)],
          [((p // NB + 1) % NA) * NB + p % NB for p in range(N)],
          [((p // NB - 1) % NA) * NB + p % NB for p in range(N)],
          [p % NB for p in range(N)],
          [(p // NB) * NB + (p + 1) % NB for p in range(N)],
          [(p // NB) * NB + (p - 1) % NB for p in range(N)])
    try:
        import distributed_mesh_v7x as dm
        mesh = dm.get_mesh("i", world_size=N)
        devs = list(mesh.devices.flat)
        coords = [tuple(d.coords) for d in devs]
        if len(set(coords)) != N or any(len(c) != 3 for c in coords):
            return fb
        axes = [sorted({c[i] for c in coords}) for i in range(3)]
        sizes = [len(a) for a in axes]
        if sorted(sizes) != [2, 4, 4]:
            return fb
        a2 = sizes.index(2)
        a4 = [i for i in range(3) if i != a2]
        us = axes[a4[0]]
        vs = axes[a4[1]]
        lo, hi = axes[a2]
        cyc = [(lo, u) for u in us] + [(hi, u) for u in reversed(us)]
        posA_of = {xu: i for i, xu in enumerate(cyc)}
        log_of = {c: p for p, c in enumerate(coords)}

        def at(c, i2, iu, iv):
            t = [0, 0, 0]
            t[a2], t[a4[0]], t[a4[1]] = i2, iu, iv
            return tuple(t)

        posA = [0] * N
        rtA = [0] * N
        ltA = [0] * N
        posB = [0] * N
        rtB = [0] * N
        ltB = [0] * N
        for p, c in enumerate(coords):
            i2, iu, iv = c[a2], c[a4[0]], c[a4[1]]
            pa = posA_of[(i2, iu)]
            pb = vs.index(iv)
            posA[p] = pa
            posB[p] = pb
            nxt = cyc[(pa + 1) % NA]
            prv = cyc[(pa - 1) % NA]
            rtA[p] = log_of[at(c, nxt[0], nxt[1], iv)]
            ltA[p] = log_of[at(c, prv[0], prv[1], iv)]
            rtB[p] = log_of[at(c, i2, iu, vs[(pb + 1) % NB])]
            ltB[p] = log_of[at(c, i2, iu, vs[(pb - 1) % NB])]
        return posA, rtA, ltA, posB, rtB, ltB
    except Exception:
        return fb


_POSA, _RTA, _LTA, _POSB, _RTB, _LTB = _build_tables()


def _body(scal_ref, x_ref, k_ref, wp_ref, out_ref,
          pad_ref, part_ref, obf_ref, bufAR, bufAL, bufBR, bufBL,
          rsA_sR, rsA_rR, rsA_sL, rsA_rL,
          rsB_sR, rsB_rR, rsB_sL, rsB_rL,
          agB_sR, agB_rR, agB_sL, agB_rL,
          agA_sR, agA_rR, agA_sL, agA_rL):
    posA = scal_ref[0]
    rtA = scal_ref[1]
    ltA = scal_ref[2]
    posB = scal_ref[3]
    rtB = scal_ref[4]
    ltB = scal_ref[5]

    barrier_sem = pltpu.get_barrier_semaphore()
    for nbr in (rtA, ltA, rtB, ltB):
        pl.semaphore_signal(barrier_sem, inc=1, device_id=(nbr,),
                            device_id_type=pl.DeviceIdType.MESH)

    kv = k_ref[:, :]
    wpv = wp_ref[:, :]
    for b in range(B):
        pad_ref[b, 0:TAPS - 1, :] = jnp.zeros((TAPS - 1, C), jnp.float32)
        pad_ref[b, TAPS - 1:TAPS - 1 + S, :] = x_ref[b]

    def compute_half(c, half):
        b = c // 2
        rl = (c % 2) * AC + half * H
        w = pad_ref[b, pl.ds(rl, H + 8), :]
        acc = w[0:H, :] * kv[0:1, :]
        for t in range(1, TAPS):
            acc = acc + w[t:t + H, :] * kv[t:t + 1, :]
        a = acc * (1.0 / (1.0 + jnp.exp(-acc)))
        part_ref[pl.ds(c * AC + half * H, H), :] = jnp.dot(
            a, wpv, preferred_element_type=jnp.float32)

    compute_half(posA % NA, 0)
    compute_half((posA - 1) % NA, 0)
    compute_half(posA % NA, 1)
    compute_half((posA + 1) % NA, 1)

    pl.semaphore_wait(barrier_sem, 4)

    def copy(src, dst, ssem, rsem, dev):
        r = pltpu.make_async_remote_copy(
            src_ref=src, dst_ref=dst, send_sem=ssem, recv_sem=rsem,
            device_id=dev, device_id_type=pl.DeviceIdType.LOGICAL)
        r.start()
        return r

    for s in range(NA - 1):
        csR = ((posA - s) % NA) * AC
        crR = ((posA - s - 1) % NA) * AC
        csL = ((posA + s) % NA) * AC + H
        crL = ((posA + s + 1) % NA) * AC + H
        rR = copy(part_ref.at[pl.ds(csR, H), :], bufAR.at[s],
                  rsA_sR.at[s], rsA_rR.at[s], rtA)
        rL = copy(part_ref.at[pl.ds(csL, H), :], bufAL.at[s],
                  rsA_sL.at[s], rsA_rL.at[s], ltA)
        if s < NA - 2:
            compute_half((posA - s - 2) % NA, 0)
            compute_half((posA + s + 2) % NA, 1)
        rR.wait_recv()
        part_ref[pl.ds(crR, H), :] = part_ref[pl.ds(crR, H), :] + bufAR[s]
        rL.wait_recv()
        part_ref[pl.ds(crL, H), :] = part_ref[pl.ds(crL, H), :] + bufAL[s]
        rR.wait_send()
        rL.wait_send()

    rbase = ((posA + 1) % NA) * AC
    lbase = ((posA - 1) % NA) * AC + H

    for s in range(NB - 1):
        sR = rbase + ((posB - s) % NB) * BC
        aR = rbase + ((posB - s - 1) % NB) * BC
        sL = lbase + ((posB + s) % NB) * BC
        aL = lbase + ((posB + s + 1) % NB) * BC
        rR = copy(part_ref.at[pl.ds(sR, BC), :], bufBR.at[s],
                  rsB_sR.at[s], rsB_rR.at[s], rtB)
        rL = copy(part_ref.at[pl.ds(sL, BC), :], bufBL.at[s],
                  rsB_sL.at[s], rsB_rL.at[s], ltB)
        rR.wait_recv()
        part_ref[pl.ds(aR, BC), :] = part_ref[pl.ds(aR, BC), :] + bufBR[s]
        rL.wait_recv()
        part_ref[pl.ds(aL, BC), :] = part_ref[pl.ds(aL, BC), :] + bufBL[s]
        rR.wait_send()
        rL.wait_send()

    ownR = rbase + ((posB + 1) % NB) * BC
    ownL = lbase + ((posB - 1) % NB) * BC
    obf_ref[pl.ds(ownR, BC), :] = part_ref[pl.ds(ownR, BC), :].astype(
        jnp.bfloat16)
    obf_ref[pl.ds(ownL, BC), :] = part_ref[pl.ds(ownL, BC), :].astype(
        jnp.bfloat16)

    for s in range(NB - 1):
        sR = rbase + ((posB + 1 - s) % NB) * BC
        sL = lbase + ((posB - 1 + s) % NB) * BC
        rR = copy(obf_ref.at[pl.ds(sR, BC), :], obf_ref.at[pl.ds(sR, BC), :],
                  agB_sR.at[s], agB_rR.at[s], rtB)
        rL = copy(obf_ref.at[pl.ds(sL, BC), :], obf_ref.at[pl.ds(sL, BC), :],
                  agB_sL.at[s], agB_rL.at[s], ltB)
        rR.wait_recv()
        rL.wait_recv()
        rR.wait_send()
        rL.wait_send()

    for s in range(NA - 1):
        sR = ((posA + 1 - s) % NA) * AC
        sL = ((posA - 1 + s) % NA) * AC + H
        rR = copy(obf_ref.at[pl.ds(sR, H), :], obf_ref.at[pl.ds(sR, H), :],
                  agA_sR.at[s], agA_rR.at[s], rtA)
        rL = copy(obf_ref.at[pl.ds(sL, H), :], obf_ref.at[pl.ds(sL, H), :],
                  agA_sL.at[s], agA_rL.at[s], ltA)
        rR.wait_recv()
        rL.wait_recv()
        rR.wait_send()
        rL.wait_send()

    for b in range(B):
        out_ref[pl.ds(b * S, S), :] = obf_ref[pl.ds(b * S, S), :].astype(
            jnp.float32)


def kernel(x, k, Wp):
    my = lax.axis_index("i")
    scalars = jnp.stack([
        jnp.asarray(_POSA, jnp.int32)[my],
        jnp.asarray(_RTA, jnp.int32)[my],
        jnp.asarray(_LTA, jnp.int32)[my],
        jnp.asarray(_POSB, jnp.int32)[my],
        jnp.asarray(_RTB, jnp.int32)[my],
        jnp.asarray(_LTB, jnp.int32)[my],
    ]).astype(jnp.int32)

    sem7 = pltpu.SemaphoreType.DMA((NA - 1,))
    sem3 = pltpu.SemaphoreType.DMA((NB - 1,))
    out = pl.pallas_call(
        _body,
        out_shape=jax.ShapeDtypeStruct((ROWS, OUT_N), jnp.float32),
        in_specs=[
            pl.BlockSpec(memory_space=pltpu.SMEM),
            pl.BlockSpec(memory_space=pltpu.VMEM),
            pl.BlockSpec(memory_space=pltpu.VMEM),
            pl.BlockSpec(memory_space=pltpu.VMEM),
        ],
        out_specs=pl.BlockSpec(memory_space=pltpu.VMEM),
        scratch_shapes=[
            pltpu.VMEM((B, S + 16, C), jnp.float32),
            pltpu.VMEM((ROWS, OUT_N), jnp.float32),
            pltpu.VMEM((ROWS, OUT_N), jnp.bfloat16),
            pltpu.VMEM((NA - 1, H, OUT_N), jnp.float32),
            pltpu.VMEM((NA - 1, H, OUT_N), jnp.float32),
            pltpu.VMEM((NB - 1, BC, OUT_N), jnp.float32),
            pltpu.VMEM((NB - 1, BC, OUT_N), jnp.float32),
            sem7, sem7, sem7, sem7,
            sem3, sem3, sem3, sem3,
            sem3, sem3, sem3, sem3,
            sem7, sem7, sem7, sem7,
        ],
        compiler_params=pltpu.CompilerParams(collective_id=0),
    )(scalars, x, k, Wp)
    return out.reshape(B, S, OUT_N)


# device time: 114006 ns/iter; 2.6506x vs baseline; 1.2006x over previous
import jax
import jax.numpy as jnp
import numpy as np
from jax import lax
from jax.experimental import pallas as pl
from jax.experimental.pallas import tpu as pltpu

N = 32
B, S, C = 4, 1024, 512
OUT_N = 512
ROWS = B * S
TAPS = 4

NA = 8
NB = 4
AC = ROWS // NA
H = AC // 2
BC = H // NB


def _build_tables():
    fb = ([p // NB for p in range(N)],
          [((p // NB + 1) % NA) * NB + p % NB for p in range(N)],
          [((p // NB - 1) % NA) * NB + p % NB for p in range(N)],
          [p % NB for p in range(N)],
          [(p // NB) * NB + (p + 1) % NB for p in range(N)],
          [(p // NB) * NB + (p - 1) % NB for p in range(N)])
    try:
        import distributed_mesh_v7x as dm
        mesh = dm.get_mesh("i", world_size=N)
        devs = list(mesh.devices.flat)
        coords = [tuple(d.coords) for d in devs]
        if len(set(coords)) != N or any(len(c) != 3 for c in coords):
            return fb
        axes = [sorted({c[i] for c in coords}) for i in range(3)]
        sizes = [len(a) for a in axes]
        if sorted(sizes) != [2, 4, 4]:
            return fb
        a2 = sizes.index(2)
        a4 = [i for i in range(3) if i != a2]
        us = axes[a4[0]]
        vs = axes[a4[1]]
        lo, hi = axes[a2]
        cyc = [(lo, u) for u in us] + [(hi, u) for u in reversed(us)]
        posA_of = {xu: i for i, xu in enumerate(cyc)}
        log_of = {c: p for p, c in enumerate(coords)}

        def at(c, i2, iu, iv):
            t = [0, 0, 0]
            t[a2], t[a4[0]], t[a4[1]] = i2, iu, iv
            return tuple(t)

        posA = [0] * N
        rtA = [0] * N
        ltA = [0] * N
        posB = [0] * N
        rtB = [0] * N
        ltB = [0] * N
        for p, c in enumerate(coords):
            i2, iu, iv = c[a2], c[a4[0]], c[a4[1]]
            pa = posA_of[(i2, iu)]
            pb = vs.index(iv)
            posA[p] = pa
            posB[p] = pb
            nxt = cyc[(pa + 1) % NA]
            prv = cyc[(pa - 1) % NA]
            rtA[p] = log_of[at(c, nxt[0], nxt[1], iv)]
            ltA[p] = log_of[at(c, prv[0], prv[1], iv)]
            rtB[p] = log_of[at(c, i2, iu, vs[(pb + 1) % NB])]
            ltB[p] = log_of[at(c, i2, iu, vs[(pb - 1) % NB])]
        return posA, rtA, ltA, posB, rtB, ltB
    except Exception:
        return fb


_POSA, _RTA, _LTA, _POSB, _RTB, _LTB = _build_tables()


def _body(scal_ref, x_ref, k_ref, wp_ref, out_ref,
          pad_ref, part_ref, obf_ref, bufAR, bufAL, bufBR, bufBL,
          sbAR, sbAL, sbBR, sbBL,
          rsA_sR, rsA_rR, rsA_sL, rsA_rL,
          rsB_sR, rsB_rR, rsB_sL, rsB_rL,
          agB_sR, agB_rR, agB_sL, agB_rL,
          agA_sR, agA_rR, agA_sL, agA_rL):
    posA = scal_ref[0]
    rtA = scal_ref[1]
    ltA = scal_ref[2]
    posB = scal_ref[3]
    rtB = scal_ref[4]
    ltB = scal_ref[5]

    barrier_sem = pltpu.get_barrier_semaphore()
    for nbr in (rtA, ltA, rtB, ltB):
        pl.semaphore_signal(barrier_sem, inc=1, device_id=(nbr,),
                            device_id_type=pl.DeviceIdType.MESH)

    kv = k_ref[:, :]
    wpv = wp_ref[:, :]
    for b in range(B):
        pad_ref[b, 0:TAPS - 1, :] = jnp.zeros((TAPS - 1, C), jnp.float32)
        pad_ref[b, TAPS - 1:TAPS - 1 + S, :] = x_ref[b]

    def compute_half(c, half):
        b = c // 2
        rl = (c % 2) * AC + half * H
        w = pad_ref[b, pl.ds(rl, H + 8), :]
        acc = w[0:H, :] * kv[0:1, :]
        for t in range(1, TAPS):
            acc = acc + w[t:t + H, :] * kv[t:t + 1, :]
        a = acc * (1.0 / (1.0 + jnp.exp(-acc)))
        part_ref[pl.ds(c * AC + half * H, H), :] = jnp.dot(
            a, wpv, preferred_element_type=jnp.float32)

    compute_half(posA % NA, 0)
    compute_half((posA - 1) % NA, 0)
    compute_half(posA % NA, 1)
    compute_half((posA + 1) % NA, 1)

    pl.semaphore_wait(barrier_sem, 4)

    def copy(src, dst, ssem, rsem, dev):
        r = pltpu.make_async_remote_copy(
            src_ref=src, dst_ref=dst, send_sem=ssem, recv_sem=rsem,
            device_id=dev, device_id_type=pl.DeviceIdType.LOGICAL)
        r.start()
        return r

    sbAR[0] = part_ref[pl.ds((posA % NA) * AC, H), :].astype(jnp.bfloat16)
    sbAL[0] = part_ref[pl.ds((posA % NA) * AC + H, H), :].astype(jnp.bfloat16)
    for s in range(NA - 1):
        crR = ((posA - s - 1) % NA) * AC
        crL = ((posA + s + 1) % NA) * AC + H
        rR = copy(sbAR.at[s], bufAR.at[s],
                  rsA_sR.at[s], rsA_rR.at[s], rtA)
        rL = copy(sbAL.at[s], bufAL.at[s],
                  rsA_sL.at[s], rsA_rL.at[s], ltA)
        if s < NA - 2:
            compute_half((posA - s - 2) % NA, 0)
            compute_half((posA + s + 2) % NA, 1)
        rR.wait_recv()
        vR = part_ref[pl.ds(crR, H), :] + bufAR[s].astype(jnp.float32)
        if s < NA - 2:
            sbAR[s + 1] = vR.astype(jnp.bfloat16)
        else:
            part_ref[pl.ds(crR, H), :] = vR
        rL.wait_recv()
        vL = part_ref[pl.ds(crL, H), :] + bufAL[s].astype(jnp.float32)
        if s < NA - 2:
            sbAL[s + 1] = vL.astype(jnp.bfloat16)
        else:
            part_ref[pl.ds(crL, H), :] = vL
        rR.wait_send()
        rL.wait_send()

    rbase = ((posA + 1) % NA) * AC
    lbase = ((posA - 1) % NA) * AC + H

    sbBR[0] = part_ref[pl.ds(rbase + (posB % NB) * BC, BC), :].astype(
        jnp.bfloat16)
    sbBL[0] = part_ref[pl.ds(lbase + (posB % NB) * BC, BC), :].astype(
        jnp.bfloat16)
    for s in range(NB - 1):
        aR = rbase + ((posB - s - 1) % NB) * BC
        aL = lbase + ((posB + s + 1) % NB) * BC
        rR = copy(sbBR.at[s], bufBR.at[s],
                  rsB_sR.at[s], rsB_rR.at[s], rtB)
        rL = copy(sbBL.at[s], bufBL.at[s],
                  rsB_sL.at[s], rsB_rL.at[s], ltB)
        rR.wait_recv()
        vR = part_ref[pl.ds(aR, BC), :] + bufBR[s].astype(jnp.float32)
        if s < NB - 2:
            sbBR[s + 1] = vR.astype(jnp.bfloat16)
        else:
            part_ref[pl.ds(aR, BC), :] = vR
        rL.wait_recv()
        vL = part_ref[pl.ds(aL, BC), :] + bufBL[s].astype(jnp.float32)
        if s < NB - 2:
            sbBL[s + 1] = vL.astype(jnp.bfloat16)
        else:
            part_ref[pl.ds(aL, BC), :] = vL
        rR.wait_send()
        rL.wait_send()

    ownR = rbase + ((posB + 1) % NB) * BC
    ownL = lbase + ((posB - 1) % NB) * BC
    obf_ref[pl.ds(ownR, BC), :] = part_ref[pl.ds(ownR, BC), :].astype(
        jnp.bfloat16)
    obf_ref[pl.ds(ownL, BC), :] = part_ref[pl.ds(ownL, BC), :].astype(
        jnp.bfloat16)

    for s in range(NB - 1):
        sR = rbase + ((posB + 1 - s) % NB) * BC
        sL = lbase + ((posB - 1 + s) % NB) * BC
        rR = copy(obf_ref.at[pl.ds(sR, BC), :], obf_ref.at[pl.ds(sR, BC), :],
                  agB_sR.at[s], agB_rR.at[s], rtB)
        rL = copy(obf_ref.at[pl.ds(sL, BC), :], obf_ref.at[pl.ds(sL, BC), :],
                  agB_sL.at[s], agB_rL.at[s], ltB)
        rR.wait_recv()
        rL.wait_recv()
        rR.wait_send()
        rL.wait_send()

    for s in range(NA - 1):
        sR = ((posA + 1 - s) % NA) * AC
        sL = ((posA - 1 + s) % NA) * AC + H
        rR = copy(obf_ref.at[pl.ds(sR, H), :], obf_ref.at[pl.ds(sR, H), :],
                  agA_sR.at[s], agA_rR.at[s], rtA)
        rL = copy(obf_ref.at[pl.ds(sL, H), :], obf_ref.at[pl.ds(sL, H), :],
                  agA_sL.at[s], agA_rL.at[s], ltA)
        rR.wait_recv()
        rL.wait_recv()
        rR.wait_send()
        rL.wait_send()

    for b in range(B):
        out_ref[pl.ds(b * S, S), :] = obf_ref[pl.ds(b * S, S), :].astype(
            jnp.float32)


def kernel(x, k, Wp):
    my = lax.axis_index("i")
    scalars = jnp.stack([
        jnp.asarray(_POSA, jnp.int32)[my],
        jnp.asarray(_RTA, jnp.int32)[my],
        jnp.asarray(_LTA, jnp.int32)[my],
        jnp.asarray(_POSB, jnp.int32)[my],
        jnp.asarray(_RTB, jnp.int32)[my],
        jnp.asarray(_LTB, jnp.int32)[my],
    ]).astype(jnp.int32)

    sem7 = pltpu.SemaphoreType.DMA((NA - 1,))
    sem3 = pltpu.SemaphoreType.DMA((NB - 1,))
    out = pl.pallas_call(
        _body,
        out_shape=jax.ShapeDtypeStruct((ROWS, OUT_N), jnp.float32),
        in_specs=[
            pl.BlockSpec(memory_space=pltpu.SMEM),
            pl.BlockSpec(memory_space=pltpu.VMEM),
            pl.BlockSpec(memory_space=pltpu.VMEM),
            pl.BlockSpec(memory_space=pltpu.VMEM),
        ],
        out_specs=pl.BlockSpec(memory_space=pltpu.VMEM),
        scratch_shapes=[
            pltpu.VMEM((B, S + 16, C), jnp.float32),
            pltpu.VMEM((ROWS, OUT_N), jnp.float32),
            pltpu.VMEM((ROWS, OUT_N), jnp.bfloat16),
            pltpu.VMEM((NA - 1, H, OUT_N), jnp.bfloat16),
            pltpu.VMEM((NA - 1, H, OUT_N), jnp.bfloat16),
            pltpu.VMEM((NB - 1, BC, OUT_N), jnp.bfloat16),
            pltpu.VMEM((NB - 1, BC, OUT_N), jnp.bfloat16),
            pltpu.VMEM((NA - 1, H, OUT_N), jnp.bfloat16),
            pltpu.VMEM((NA - 1, H, OUT_N), jnp.bfloat16),
            pltpu.VMEM((NB - 1, BC, OUT_N), jnp.bfloat16),
            pltpu.VMEM((NB - 1, BC, OUT_N), jnp.bfloat16),
            sem7, sem7, sem7, sem7,
            sem3, sem3, sem3, sem3,
            sem3, sem3, sem3, sem3,
            sem7, sem7, sem7, sem7,
        ],
        compiler_params=pltpu.CompilerParams(collective_id=0),
    )(scalars, x, k, Wp)
    return out.reshape(B, S, OUT_N)


# device time: 100178 ns/iter; 3.0164x vs baseline; 1.1380x over previous
import jax
import jax.numpy as jnp
import numpy as np
from jax import lax
from jax.experimental import pallas as pl
from jax.experimental.pallas import tpu as pltpu

N = 32
B, S, C = 4, 1024, 512
OUT_N = 512
ROWS = B * S
TAPS = 4

NA = 8
NB = 4
AC = ROWS // NA
H = AC // 2
QH = H // 2
BC = H // NB


def _build_tables():
    fb = ([p // NB for p in range(N)],
          [((p // NB + 1) % NA) * NB + p % NB for p in range(N)],
          [((p // NB - 1) % NA) * NB + p % NB for p in range(N)],
          [p % NB for p in range(N)],
          [(p // NB) * NB + (p + 1) % NB for p in range(N)],
          [(p // NB) * NB + (p - 1) % NB for p in range(N)])
    try:
        import distributed_mesh_v7x as dm
        mesh = dm.get_mesh("i", world_size=N)
        devs = list(mesh.devices.flat)
        coords = [tuple(d.coords) for d in devs]
        if len(set(coords)) != N or any(len(c) != 3 for c in coords):
            return fb
        axes = [sorted({c[i] for c in coords}) for i in range(3)]
        sizes = [len(a) for a in axes]
        if sorted(sizes) != [2, 4, 4]:
            return fb
        a2 = sizes.index(2)
        a4 = [i for i in range(3) if i != a2]
        us = axes[a4[0]]
        vs = axes[a4[1]]
        lo, hi = axes[a2]
        cyc = [(lo, u) for u in us] + [(hi, u) for u in reversed(us)]
        posA_of = {xu: i for i, xu in enumerate(cyc)}
        log_of = {c: p for p, c in enumerate(coords)}

        def at(c, i2, iu, iv):
            t = [0, 0, 0]
            t[a2], t[a4[0]], t[a4[1]] = i2, iu, iv
            return tuple(t)

        posA = [0] * N
        rtA = [0] * N
        ltA = [0] * N
        posB = [0] * N
        rtB = [0] * N
        ltB = [0] * N
        for p, c in enumerate(coords):
            i2, iu, iv = c[a2], c[a4[0]], c[a4[1]]
            pa = posA_of[(i2, iu)]
            pb = vs.index(iv)
            posA[p] = pa
            posB[p] = pb
            nxt = cyc[(pa + 1) % NA]
            prv = cyc[(pa - 1) % NA]
            rtA[p] = log_of[at(c, nxt[0], nxt[1], iv)]
            ltA[p] = log_of[at(c, prv[0], prv[1], iv)]
            rtB[p] = log_of[at(c, i2, iu, vs[(pb + 1) % NB])]
            ltB[p] = log_of[at(c, i2, iu, vs[(pb - 1) % NB])]
        return posA, rtA, ltA, posB, rtB, ltB
    except Exception:
        return fb


_POSA, _RTA, _LTA, _POSB, _RTB, _LTB = _build_tables()


def _body(scal_ref, x_ref, k_ref, wp_ref, out_ref,
          pad_ref, part_ref, obf_ref, bufAR, bufAL, bufBR, bufBL,
          sbAR, sbAL, sbBR, sbBL,
          rsA_sR, rsA_rR, rsA_sL, rsA_rL,
          rsB_sR, rsB_rR, rsB_sL, rsB_rL,
          agB_sR, agB_rR, agB_sL, agB_rL,
          agA_sR, agA_rR, agA_sL, agA_rL):
    posA = scal_ref[0]
    rtA = scal_ref[1]
    ltA = scal_ref[2]
    posB = scal_ref[3]
    rtB = scal_ref[4]
    ltB = scal_ref[5]

    barrier_sem = pltpu.get_barrier_semaphore()
    for nbr in (rtA, ltA, rtB, ltB):
        pl.semaphore_signal(barrier_sem, inc=1, device_id=(nbr,),
                            device_id_type=pl.DeviceIdType.MESH)

    kv = k_ref[:, :]
    wpv = wp_ref[:, :]
    for b in range(B):
        pad_ref[b, 0:TAPS - 1, :] = jnp.zeros((TAPS - 1, C), jnp.float32)
        pad_ref[b, TAPS - 1:TAPS - 1 + S, :] = x_ref[b]

    def compute_half(c, half):
        b = c // 2
        rl = (c % 2) * AC + half * H
        w = pad_ref[b, pl.ds(rl, H + 8), :]
        acc = w[0:H, :] * kv[0:1, :]
        for t in range(1, TAPS):
            acc = acc + w[t:t + H, :] * kv[t:t + 1, :]
        a = acc * (1.0 / (1.0 + jnp.exp(-acc)))
        part_ref[pl.ds(c * AC + half * H, H), :] = jnp.dot(
            a, wpv, preferred_element_type=jnp.float32)

    compute_half(posA % NA, 0)
    compute_half((posA - 1) % NA, 0)
    compute_half(posA % NA, 1)
    compute_half((posA + 1) % NA, 1)

    pl.semaphore_wait(barrier_sem, 4)

    def copy(src, dst, ssem, rsem, dev):
        r = pltpu.make_async_remote_copy(
            src_ref=src, dst_ref=dst, send_sem=ssem, recv_sem=rsem,
            device_id=dev, device_id_type=pl.DeviceIdType.LOGICAL)
        r.start()
        return r

    sbAR[0] = part_ref[pl.ds((posA % NA) * AC, H), :].astype(jnp.bfloat16)
    sbAL[0] = part_ref[pl.ds((posA % NA) * AC + H, H), :].astype(jnp.bfloat16)

    def rsA_send(d, q, s):
        sb, buf, ss, rs, dev = (
            (sbAR, bufAR, rsA_sR, rsA_rR, rtA) if d == 0 else
            (sbAL, bufAL, rsA_sL, rsA_rL, ltA))
        return copy(sb.at[s, pl.ds(q * QH, QH), :],
                    buf.at[s, pl.ds(q * QH, QH), :],
                    ss.at[2 * s + q], rs.at[2 * s + q], dev)

    dR = [[rsA_send(0, q, 0) for q in (0, 1)]]
    dL = [[rsA_send(1, q, 0) for q in (0, 1)]]
    for s in range(NA - 1):
        crR = ((posA - s - 1) % NA) * AC
        crL = ((posA + s + 1) % NA) * AC + H
        if s + 1 < NA - 1:
            dR.append([None, None])
            dL.append([None, None])
        for q in (0, 1):
            dR[s][q].wait_recv()
            rows = pl.ds(crR + q * QH, QH)
            v = part_ref[rows, :] + bufAR[s, q * QH:(q + 1) * QH, :].astype(
                jnp.float32)
            if s < NA - 2:
                sbAR[s + 1, q * QH:(q + 1) * QH, :] = v.astype(jnp.bfloat16)
                dR[s + 1][q] = rsA_send(0, q, s + 1)
            else:
                part_ref[rows, :] = v
        for q in (0, 1):
            dL[s][q].wait_recv()
            rows = pl.ds(crL + q * QH, QH)
            v = part_ref[rows, :] + bufAL[s, q * QH:(q + 1) * QH, :].astype(
                jnp.float32)
            if s < NA - 2:
                sbAL[s + 1, q * QH:(q + 1) * QH, :] = v.astype(jnp.bfloat16)
                dL[s + 1][q] = rsA_send(1, q, s + 1)
            else:
                part_ref[rows, :] = v
        if s < NA - 2:
            compute_half((posA - s - 2) % NA, 0)
            compute_half((posA + s + 2) % NA, 1)
        for q in (0, 1):
            dR[s][q].wait_send()
            dL[s][q].wait_send()

    rbase = ((posA + 1) % NA) * AC
    lbase = ((posA - 1) % NA) * AC + H

    sbBR[0] = part_ref[pl.ds(rbase + (posB % NB) * BC, BC), :].astype(
        jnp.bfloat16)
    sbBL[0] = part_ref[pl.ds(lbase + (posB % NB) * BC, BC), :].astype(
        jnp.bfloat16)
    for s in range(NB - 1):
        aR = rbase + ((posB - s - 1) % NB) * BC
        aL = lbase + ((posB + s + 1) % NB) * BC
        rR = copy(sbBR.at[s], bufBR.at[s],
                  rsB_sR.at[s], rsB_rR.at[s], rtB)
        rL = copy(sbBL.at[s], bufBL.at[s],
                  rsB_sL.at[s], rsB_rL.at[s], ltB)
        rR.wait_recv()
        vR = part_ref[pl.ds(aR, BC), :] + bufBR[s].astype(jnp.float32)
        if s < NB - 2:
            sbBR[s + 1] = vR.astype(jnp.bfloat16)
        else:
            part_ref[pl.ds(aR, BC), :] = vR
        rL.wait_recv()
        vL = part_ref[pl.ds(aL, BC), :] + bufBL[s].astype(jnp.float32)
        if s < NB - 2:
            sbBL[s + 1] = vL.astype(jnp.bfloat16)
        else:
            part_ref[pl.ds(aL, BC), :] = vL
        rR.wait_send()
        rL.wait_send()

    ownR = rbase + ((posB + 1) % NB) * BC
    ownL = lbase + ((posB - 1) % NB) * BC
    obf_ref[pl.ds(ownR, BC), :] = part_ref[pl.ds(ownR, BC), :].astype(
        jnp.bfloat16)
    obf_ref[pl.ds(ownL, BC), :] = part_ref[pl.ds(ownL, BC), :].astype(
        jnp.bfloat16)

    for s in range(NB - 1):
        sR = rbase + ((posB + 1 - s) % NB) * BC
        sL = lbase + ((posB - 1 + s) % NB) * BC
        rR = copy(obf_ref.at[pl.ds(sR, BC), :], obf_ref.at[pl.ds(sR, BC), :],
                  agB_sR.at[s], agB_rR.at[s], rtB)
        rL = copy(obf_ref.at[pl.ds(sL, BC), :], obf_ref.at[pl.ds(sL, BC), :],
                  agB_sL.at[s], agB_rL.at[s], ltB)
        rR.wait_recv()
        rL.wait_recv()
        rR.wait_send()
        rL.wait_send()

    def agA_send(d, q, s):
        if d == 0:
            rows = pl.ds(((posA + 1 - s) % NA) * AC + q * QH, QH)
            ss, rs, dev = agA_sR, agA_rR, rtA
        else:
            rows = pl.ds(((posA - 1 + s) % NA) * AC + H + q * QH, QH)
            ss, rs, dev = agA_sL, agA_rL, ltA
        return copy(obf_ref.at[rows, :], obf_ref.at[rows, :],
                    ss.at[2 * s + q], rs.at[2 * s + q], dev)

    gR = [[agA_send(0, q, 0) for q in (0, 1)]]
    gL = [[agA_send(1, q, 0) for q in (0, 1)]]
    for s in range(NA - 1):
        if s + 1 < NA - 1:
            gR.append([None, None])
            gL.append([None, None])
        for q in (0, 1):
            gR[s][q].wait_recv()
            if s + 1 < NA - 1:
                gR[s + 1][q] = agA_send(0, q, s + 1)
        for q in (0, 1):
            gL[s][q].wait_recv()
            if s + 1 < NA - 1:
                gL[s + 1][q] = agA_send(1, q, s + 1)
        for q in (0, 1):
            gR[s][q].wait_send()
            gL[s][q].wait_send()

    for b in range(B):
        out_ref[pl.ds(b * S, S), :] = obf_ref[pl.ds(b * S, S), :].astype(
            jnp.float32)


def kernel(x, k, Wp):
    my = lax.axis_index("i")
    scalars = jnp.stack([
        jnp.asarray(_POSA, jnp.int32)[my],
        jnp.asarray(_RTA, jnp.int32)[my],
        jnp.asarray(_LTA, jnp.int32)[my],
        jnp.asarray(_POSB, jnp.int32)[my],
        jnp.asarray(_RTB, jnp.int32)[my],
        jnp.asarray(_LTB, jnp.int32)[my],
    ]).astype(jnp.int32)

    sem7 = pltpu.SemaphoreType.DMA((2 * (NA - 1),))
    sem3 = pltpu.SemaphoreType.DMA((NB - 1,))
    out = pl.pallas_call(
        _body,
        out_shape=jax.ShapeDtypeStruct((ROWS, OUT_N), jnp.float32),
        in_specs=[
            pl.BlockSpec(memory_space=pltpu.SMEM),
            pl.BlockSpec(memory_space=pltpu.VMEM),
            pl.BlockSpec(memory_space=pltpu.VMEM),
            pl.BlockSpec(memory_space=pltpu.VMEM),
        ],
        out_specs=pl.BlockSpec(memory_space=pltpu.VMEM),
        scratch_shapes=[
            pltpu.VMEM((B, S + 16, C), jnp.float32),
            pltpu.VMEM((ROWS, OUT_N), jnp.float32),
            pltpu.VMEM((ROWS, OUT_N), jnp.bfloat16),
            pltpu.VMEM((NA - 1, H, OUT_N), jnp.bfloat16),
            pltpu.VMEM((NA - 1, H, OUT_N), jnp.bfloat16),
            pltpu.VMEM((NB - 1, BC, OUT_N), jnp.bfloat16),
            pltpu.VMEM((NB - 1, BC, OUT_N), jnp.bfloat16),
            pltpu.VMEM((NA - 1, H, OUT_N), jnp.bfloat16),
            pltpu.VMEM((NA - 1, H, OUT_N), jnp.bfloat16),
            pltpu.VMEM((NB - 1, BC, OUT_N), jnp.bfloat16),
            pltpu.VMEM((NB - 1, BC, OUT_N), jnp.bfloat16),
            sem7, sem7, sem7, sem7,
            sem3, sem3, sem3, sem3,
            sem3, sem3, sem3, sem3,
            sem7, sem7, sem7, sem7,
        ],
        compiler_params=pltpu.CompilerParams(collective_id=0),
    )(scalars, x, k, Wp)
    return out.reshape(B, S, OUT_N)


# device time: 98629 ns/iter; 3.0638x vs baseline; 1.0157x over previous
import jax
import jax.numpy as jnp
import numpy as np
from jax import lax
from jax.experimental import pallas as pl
from jax.experimental.pallas import tpu as pltpu

N = 32
B, S, C = 4, 1024, 512
OUT_N = 512
ROWS = B * S
TAPS = 4

NA = 8
NB = 4
AC = ROWS // NA
H = AC // 2
QH = H // 2
BC = H // NB


def _build_tables():
    fb = ([p // NB for p in range(N)],
          [((p // NB + 1) % NA) * NB + p % NB for p in range(N)],
          [((p // NB - 1) % NA) * NB + p % NB for p in range(N)],
          [p % NB for p in range(N)],
          [(p // NB) * NB + (p + 1) % NB for p in range(N)],
          [(p // NB) * NB + (p - 1) % NB for p in range(N)])
    try:
        import distributed_mesh_v7x as dm
        mesh = dm.get_mesh("i", world_size=N)
        devs = list(mesh.devices.flat)
        coords = [tuple(d.coords) for d in devs]
        if len(set(coords)) != N or any(len(c) != 3 for c in coords):
            return fb
        axes = [sorted({c[i] for c in coords}) for i in range(3)]
        sizes = [len(a) for a in axes]
        if sorted(sizes) != [2, 4, 4]:
            return fb
        a2 = sizes.index(2)
        a4 = [i for i in range(3) if i != a2]
        us = axes[a4[0]]
        _v = axes[a4[1]]
        vs = [_v[0], _v[1], _v[3], _v[2]]
        lo, hi = axes[a2]
        cyc = [(lo, u) for u in us] + [(hi, u) for u in reversed(us)]
        posA_of = {xu: i for i, xu in enumerate(cyc)}
        log_of = {c: p for p, c in enumerate(coords)}

        def at(c, i2, iu, iv):
            t = [0, 0, 0]
            t[a2], t[a4[0]], t[a4[1]] = i2, iu, iv
            return tuple(t)

        posA = [0] * N
        rtA = [0] * N
        ltA = [0] * N
        posB = [0] * N
        rtB = [0] * N
        ltB = [0] * N
        for p, c in enumerate(coords):
            i2, iu, iv = c[a2], c[a4[0]], c[a4[1]]
            pa = posA_of[(i2, iu)]
            pb = vs.index(iv)
            posA[p] = pa
            posB[p] = pb
            nxt = cyc[(pa + 1) % NA]
            prv = cyc[(pa - 1) % NA]
            rtA[p] = log_of[at(c, nxt[0], nxt[1], iv)]
            ltA[p] = log_of[at(c, prv[0], prv[1], iv)]
            rtB[p] = log_of[at(c, i2, iu, vs[(pb + 1) % NB])]
            ltB[p] = log_of[at(c, i2, iu, vs[(pb - 1) % NB])]
        return posA, rtA, ltA, posB, rtB, ltB
    except Exception:
        return fb


_POSA, _RTA, _LTA, _POSB, _RTB, _LTB = _build_tables()


def _body(scal_ref, x_ref, k_ref, wp_ref, out_ref,
          pad_ref, part_ref, obf_ref, bufAR, bufAL, bufBR, bufBL,
          sbAR, sbAL, sbBR, sbBL,
          rsA_sR, rsA_rR, rsA_sL, rsA_rL,
          rsB_sR, rsB_rR, rsB_sL, rsB_rL,
          agB_sR, agB_rR, agB_sL, agB_rL,
          agA_sR, agA_rR, agA_sL, agA_rL):
    posA = scal_ref[0]
    rtA = scal_ref[1]
    ltA = scal_ref[2]
    posB = scal_ref[3]
    rtB = scal_ref[4]
    ltB = scal_ref[5]

    barrier_sem = pltpu.get_barrier_semaphore()
    for nbr in (rtA, ltA, rtB, ltB):
        pl.semaphore_signal(barrier_sem, inc=1, device_id=(nbr,),
                            device_id_type=pl.DeviceIdType.MESH)

    kv = k_ref[:, :]
    wpv = wp_ref[:, :]
    for b in range(B):
        pad_ref[b, 0:TAPS - 1, :] = jnp.zeros((TAPS - 1, C), jnp.float32)
        pad_ref[b, TAPS - 1:TAPS - 1 + S, :] = x_ref[b]

    def compute_half(c, half):
        b = c // 2
        rl = (c % 2) * AC + half * H
        w = pad_ref[b, pl.ds(rl, H + 8), :]
        acc = w[0:H, :] * kv[0:1, :]
        for t in range(1, TAPS):
            acc = acc + w[t:t + H, :] * kv[t:t + 1, :]
        a = acc * (1.0 / (1.0 + jnp.exp(-acc)))
        part_ref[pl.ds(c * AC + half * H, H), :] = jnp.dot(
            a, wpv, preferred_element_type=jnp.float32)

    compute_half(posA % NA, 0)
    compute_half(posA % NA, 1)

    pl.semaphore_wait(barrier_sem, 4)

    def copy(src, dst, ssem, rsem, dev):
        r = pltpu.make_async_remote_copy(
            src_ref=src, dst_ref=dst, send_sem=ssem, recv_sem=rsem,
            device_id=dev, device_id_type=pl.DeviceIdType.LOGICAL)
        r.start()
        return r

    sbAR[0] = part_ref[pl.ds((posA % NA) * AC, H), :].astype(jnp.bfloat16)
    sbAL[0] = part_ref[pl.ds((posA % NA) * AC + H, H), :].astype(jnp.bfloat16)

    def rsA_send(d, q, s):
        sb, buf, ss, rs, dev = (
            (sbAR, bufAR, rsA_sR, rsA_rR, rtA) if d == 0 else
            (sbAL, bufAL, rsA_sL, rsA_rL, ltA))
        return copy(sb.at[s, pl.ds(q * QH, QH), :],
                    buf.at[s, pl.ds(q * QH, QH), :],
                    ss.at[2 * s + q], rs.at[2 * s + q], dev)

    dR = [[rsA_send(0, q, 0) for q in (0, 1)]]
    dL = [[rsA_send(1, q, 0) for q in (0, 1)]]
    compute_half((posA - 1) % NA, 0)
    compute_half((posA + 1) % NA, 1)
    for s in range(NA - 1):
        crR = ((posA - s - 1) % NA) * AC
        crL = ((posA + s + 1) % NA) * AC + H
        if s + 1 < NA - 1:
            dR.append([None, None])
            dL.append([None, None])
        for q in (0, 1):
            dR[s][q].wait_recv()
            rows = pl.ds(crR + q * QH, QH)
            v = part_ref[rows, :] + bufAR[s, q * QH:(q + 1) * QH, :].astype(
                jnp.float32)
            if s < NA - 2:
                sbAR[s + 1, q * QH:(q + 1) * QH, :] = v.astype(jnp.bfloat16)
                dR[s + 1][q] = rsA_send(0, q, s + 1)
            else:
                part_ref[rows, :] = v
        for q in (0, 1):
            dL[s][q].wait_recv()
            rows = pl.ds(crL + q * QH, QH)
            v = part_ref[rows, :] + bufAL[s, q * QH:(q + 1) * QH, :].astype(
                jnp.float32)
            if s < NA - 2:
                sbAL[s + 1, q * QH:(q + 1) * QH, :] = v.astype(jnp.bfloat16)
                dL[s + 1][q] = rsA_send(1, q, s + 1)
            else:
                part_ref[rows, :] = v
        if s < NA - 2:
            compute_half((posA - s - 2) % NA, 0)
            compute_half((posA + s + 2) % NA, 1)
        for q in (0, 1):
            dR[s][q].wait_send()
            dL[s][q].wait_send()

    rbase = ((posA + 1) % NA) * AC
    lbase = ((posA - 1) % NA) * AC + H

    sbBR[0] = part_ref[pl.ds(rbase + (posB % NB) * BC, BC), :].astype(
        jnp.bfloat16)
    sbBL[0] = part_ref[pl.ds(lbase + (posB % NB) * BC, BC), :].astype(
        jnp.bfloat16)
    for s in range(NB - 1):
        aR = rbase + ((posB - s - 1) % NB) * BC
        aL = lbase + ((posB + s + 1) % NB) * BC
        rR = copy(sbBR.at[s], bufBR.at[s],
                  rsB_sR.at[s], rsB_rR.at[s], rtB)
        rL = copy(sbBL.at[s], bufBL.at[s],
                  rsB_sL.at[s], rsB_rL.at[s], ltB)
        rR.wait_recv()
        vR = part_ref[pl.ds(aR, BC), :] + bufBR[s].astype(jnp.float32)
        if s < NB - 2:
            sbBR[s + 1] = vR.astype(jnp.bfloat16)
        else:
            part_ref[pl.ds(aR, BC), :] = vR
        rL.wait_recv()
        vL = part_ref[pl.ds(aL, BC), :] + bufBL[s].astype(jnp.float32)
        if s < NB - 2:
            sbBL[s + 1] = vL.astype(jnp.bfloat16)
        else:
            part_ref[pl.ds(aL, BC), :] = vL
        rR.wait_send()
        rL.wait_send()

    ownR = rbase + ((posB + 1) % NB) * BC
    ownL = lbase + ((posB - 1) % NB) * BC
    obf_ref[pl.ds(ownR, BC), :] = part_ref[pl.ds(ownR, BC), :].astype(
        jnp.bfloat16)
    obf_ref[pl.ds(ownL, BC), :] = part_ref[pl.ds(ownL, BC), :].astype(
        jnp.bfloat16)

    for s in range(NB - 1):
        sR = rbase + ((posB + 1 - s) % NB) * BC
        sL = lbase + ((posB - 1 + s) % NB) * BC
        rR = copy(obf_ref.at[pl.ds(sR, BC), :], obf_ref.at[pl.ds(sR, BC), :],
                  agB_sR.at[s], agB_rR.at[s], rtB)
        rL = copy(obf_ref.at[pl.ds(sL, BC), :], obf_ref.at[pl.ds(sL, BC), :],
                  agB_sL.at[s], agB_rL.at[s], ltB)
        rR.wait_recv()
        rL.wait_recv()
        rR.wait_send()
        rL.wait_send()

    def agA_send(d, q, s):
        if d == 0:
            rows = pl.ds(((posA + 1 - s) % NA) * AC + q * QH, QH)
            ss, rs, dev = agA_sR, agA_rR, rtA
        else:
            rows = pl.ds(((posA - 1 + s) % NA) * AC + H + q * QH, QH)
            ss, rs, dev = agA_sL, agA_rL, ltA
        return copy(obf_ref.at[rows, :], obf_ref.at[rows, :],
                    ss.at[2 * s + q], rs.at[2 * s + q], dev)

    gR = [[agA_send(0, q, 0) for q in (0, 1)]]
    gL = [[agA_send(1, q, 0) for q in (0, 1)]]
    for s in range(NA - 1):
        if s + 1 < NA - 1:
            gR.append([None, None])
            gL.append([None, None])
        for q in (0, 1):
            gR[s][q].wait_recv()
            if s + 1 < NA - 1:
                gR[s + 1][q] = agA_send(0, q, s + 1)
        for q in (0, 1):
            gL[s][q].wait_recv()
            if s + 1 < NA - 1:
                gL[s + 1][q] = agA_send(1, q, s + 1)
        for q in (0, 1):
            gR[s][q].wait_send()
            gL[s][q].wait_send()

    for b in range(B):
        out_ref[pl.ds(b * S, S), :] = obf_ref[pl.ds(b * S, S), :].astype(
            jnp.float32)


def kernel(x, k, Wp):
    my = lax.axis_index("i")
    scalars = jnp.stack([
        jnp.asarray(_POSA, jnp.int32)[my],
        jnp.asarray(_RTA, jnp.int32)[my],
        jnp.asarray(_LTA, jnp.int32)[my],
        jnp.asarray(_POSB, jnp.int32)[my],
        jnp.asarray(_RTB, jnp.int32)[my],
        jnp.asarray(_LTB, jnp.int32)[my],
    ]).astype(jnp.int32)

    sem7 = pltpu.SemaphoreType.DMA((2 * (NA - 1),))
    sem3 = pltpu.SemaphoreType.DMA((NB - 1,))
    out = pl.pallas_call(
        _body,
        out_shape=jax.ShapeDtypeStruct((ROWS, OUT_N), jnp.float32),
        in_specs=[
            pl.BlockSpec(memory_space=pltpu.SMEM),
            pl.BlockSpec(memory_space=pltpu.VMEM),
            pl.BlockSpec(memory_space=pltpu.VMEM),
            pl.BlockSpec(memory_space=pltpu.VMEM),
        ],
        out_specs=pl.BlockSpec(memory_space=pltpu.VMEM),
        scratch_shapes=[
            pltpu.VMEM((B, S + 16, C), jnp.float32),
            pltpu.VMEM((ROWS, OUT_N), jnp.float32),
            pltpu.VMEM((ROWS, OUT_N), jnp.bfloat16),
            pltpu.VMEM((NA - 1, H, OUT_N), jnp.bfloat16),
            pltpu.VMEM((NA - 1, H, OUT_N), jnp.bfloat16),
            pltpu.VMEM((NB - 1, BC, OUT_N), jnp.bfloat16),
            pltpu.VMEM((NB - 1, BC, OUT_N), jnp.bfloat16),
            pltpu.VMEM((NA - 1, H, OUT_N), jnp.bfloat16),
            pltpu.VMEM((NA - 1, H, OUT_N), jnp.bfloat16),
            pltpu.VMEM((NB - 1, BC, OUT_N), jnp.bfloat16),
            pltpu.VMEM((NB - 1, BC, OUT_N), jnp.bfloat16),
            sem7, sem7, sem7, sem7,
            sem3, sem3, sem3, sem3,
            sem3, sem3, sem3, sem3,
            sem7, sem7, sem7, sem7,
        ],
        compiler_params=pltpu.CompilerParams(collective_id=0),
    )(scalars, x, k, Wp)
    return out.reshape(B, S, OUT_N)


# device time: 88171 ns/iter; 3.4272x vs baseline; 1.1186x over previous
import jax
import jax.numpy as jnp
import numpy as np
from jax import lax
from jax.experimental import pallas as pl
from jax.experimental.pallas import tpu as pltpu

N = 32
B, S, C = 4, 1024, 512
OUT_N = 512
ROWS = B * S
TAPS = 4

NA = 8
NB = 4
AC = ROWS // NA
H = AC // 2
QH = H // 2
BC = H // NB


def _build_tables():
    fb = ([p // NB for p in range(N)],
          [((p // NB + 1) % NA) * NB + p % NB for p in range(N)],
          [((p // NB - 1) % NA) * NB + p % NB for p in range(N)],
          [p % NB for p in range(N)],
          [(p // NB) * NB + (p + 1) % NB for p in range(N)],
          [(p // NB) * NB + (p - 1) % NB for p in range(N)])
    try:
        import distributed_mesh_v7x as dm
        mesh = dm.get_mesh("i", world_size=N)
        devs = list(mesh.devices.flat)
        coords = [tuple(d.coords) for d in devs]
        if len(set(coords)) != N or any(len(c) != 3 for c in coords):
            return fb
        axes = [sorted({c[i] for c in coords}) for i in range(3)]
        sizes = [len(a) for a in axes]
        if sorted(sizes) != [2, 4, 4]:
            return fb
        a2 = sizes.index(2)
        a4 = [i for i in range(3) if i != a2]
        us = axes[a4[0]]
        _v = axes[a4[1]]
        vs = [_v[0], _v[1], _v[3], _v[2]]
        lo, hi = axes[a2]
        cyc = [(lo, u) for u in us] + [(hi, u) for u in reversed(us)]
        posA_of = {xu: i for i, xu in enumerate(cyc)}
        log_of = {c: p for p, c in enumerate(coords)}

        def at(c, i2, iu, iv):
            t = [0, 0, 0]
            t[a2], t[a4[0]], t[a4[1]] = i2, iu, iv
            return tuple(t)

        posA = [0] * N
        rtA = [0] * N
        ltA = [0] * N
        posB = [0] * N
        rtB = [0] * N
        ltB = [0] * N
        for p, c in enumerate(coords):
            i2, iu, iv = c[a2], c[a4[0]], c[a4[1]]
            pa = posA_of[(i2, iu)]
            pb = vs.index(iv)
            posA[p] = pa
            posB[p] = pb
            nxt = cyc[(pa + 1) % NA]
            prv = cyc[(pa - 1) % NA]
            rtA[p] = log_of[at(c, nxt[0], nxt[1], iv)]
            ltA[p] = log_of[at(c, prv[0], prv[1], iv)]
            rtB[p] = log_of[at(c, i2, iu, vs[(pb + 1) % NB])]
            ltB[p] = log_of[at(c, i2, iu, vs[(pb - 1) % NB])]
        return posA, rtA, ltA, posB, rtB, ltB
    except Exception:
        return fb


_POSA, _RTA, _LTA, _POSB, _RTB, _LTB = _build_tables()


def _body(scal_ref, x_ref, k_ref, wp_ref, out_ref,
          pad_ref, part_ref, obf_ref, bufAR, bufAL, bufBR, bufBL,
          sbAR, sbAL, sbBR, sbBL,
          rsA_sR, rsA_rR, rsA_sL, rsA_rL,
          rsB_sR, rsB_rR, rsB_sL, rsB_rL,
          agB_sR, agB_rR, agB_sL, agB_rL,
          agA_sR, agA_rR, agA_sL, agA_rL):
    my = lax.axis_index("i")
    posA = scal_ref[0, my]
    rtA = scal_ref[1, my]
    ltA = scal_ref[2, my]
    posB = scal_ref[3, my]
    rtB = scal_ref[4, my]
    ltB = scal_ref[5, my]

    barrier_sem = pltpu.get_barrier_semaphore()
    for nbr in (rtA, ltA, rtB, ltB):
        pl.semaphore_signal(barrier_sem, inc=1, device_id=(nbr,),
                            device_id_type=pl.DeviceIdType.MESH)

    kv = k_ref[:, :]
    wpv = wp_ref[:, :]
    for b in range(B):
        pad_ref[b, 0:TAPS - 1, :] = jnp.zeros((TAPS - 1, C), jnp.float32)
        pad_ref[b, TAPS - 1:TAPS - 1 + S, :] = x_ref[b]

    def compute_half(c, half):
        b = c // 2
        rl = (c % 2) * AC + half * H
        w = pad_ref[b, pl.ds(rl, H + 8), :]
        acc = w[0:H, :] * kv[0:1, :]
        for t in range(1, TAPS):
            acc = acc + w[t:t + H, :] * kv[t:t + 1, :]
        a = acc * (1.0 / (1.0 + jnp.exp(-acc)))
        part_ref[pl.ds(c * AC + half * H, H), :] = jnp.dot(
            a, wpv, preferred_element_type=jnp.float32)

    compute_half(posA % NA, 0)
    compute_half(posA % NA, 1)

    pl.semaphore_wait(barrier_sem, 4)

    def copy(src, dst, ssem, rsem, dev):
        r = pltpu.make_async_remote_copy(
            src_ref=src, dst_ref=dst, send_sem=ssem, recv_sem=rsem,
            device_id=dev, device_id_type=pl.DeviceIdType.LOGICAL)
        r.start()
        return r

    sbAR[0] = part_ref[pl.ds((posA % NA) * AC, H), :].astype(jnp.bfloat16)
    sbAL[0] = part_ref[pl.ds((posA % NA) * AC + H, H), :].astype(jnp.bfloat16)

    def rsA_send(d, q, s):
        sb, buf, ss, rs, dev = (
            (sbAR, bufAR, rsA_sR, rsA_rR, rtA) if d == 0 else
            (sbAL, bufAL, rsA_sL, rsA_rL, ltA))
        return copy(sb.at[s, pl.ds(q * QH, QH), :],
                    buf.at[s, pl.ds(q * QH, QH), :],
                    ss.at[2 * s + q], rs.at[2 * s + q], dev)

    dR = [[rsA_send(0, q, 0) for q in (0, 1)]]
    dL = [[rsA_send(1, q, 0) for q in (0, 1)]]
    compute_half((posA - 1) % NA, 0)
    compute_half((posA + 1) % NA, 1)
    for s in range(NA - 1):
        crR = ((posA - s - 1) % NA) * AC
        crL = ((posA + s + 1) % NA) * AC + H
        if s + 1 < NA - 1:
            dR.append([None, None])
            dL.append([None, None])
        for q in (0, 1):
            dR[s][q].wait_recv()
            rows = pl.ds(crR + q * QH, QH)
            v = part_ref[rows, :] + bufAR[s, q * QH:(q + 1) * QH, :].astype(
                jnp.float32)
            if s < NA - 2:
                sbAR[s + 1, q * QH:(q + 1) * QH, :] = v.astype(jnp.bfloat16)
                dR[s + 1][q] = rsA_send(0, q, s + 1)
            else:
                part_ref[rows, :] = v
        for q in (0, 1):
            dL[s][q].wait_recv()
            rows = pl.ds(crL + q * QH, QH)
            v = part_ref[rows, :] + bufAL[s, q * QH:(q + 1) * QH, :].astype(
                jnp.float32)
            if s < NA - 2:
                sbAL[s + 1, q * QH:(q + 1) * QH, :] = v.astype(jnp.bfloat16)
                dL[s + 1][q] = rsA_send(1, q, s + 1)
            else:
                part_ref[rows, :] = v
        if s < NA - 2:
            compute_half((posA - s - 2) % NA, 0)
            compute_half((posA + s + 2) % NA, 1)
        for q in (0, 1):
            dR[s][q].wait_send()
            dL[s][q].wait_send()

    rbase = ((posA + 1) % NA) * AC
    lbase = ((posA - 1) % NA) * AC + H

    sbBR[0] = part_ref[pl.ds(rbase + (posB % NB) * BC, BC), :].astype(
        jnp.bfloat16)
    sbBL[0] = part_ref[pl.ds(lbase + (posB % NB) * BC, BC), :].astype(
        jnp.bfloat16)
    for s in range(NB - 1):
        aR = rbase + ((posB - s - 1) % NB) * BC
        aL = lbase + ((posB + s + 1) % NB) * BC
        rR = copy(sbBR.at[s], bufBR.at[s],
                  rsB_sR.at[s], rsB_rR.at[s], rtB)
        rL = copy(sbBL.at[s], bufBL.at[s],
                  rsB_sL.at[s], rsB_rL.at[s], ltB)
        rR.wait_recv()
        vR = part_ref[pl.ds(aR, BC), :] + bufBR[s].astype(jnp.float32)
        if s < NB - 2:
            sbBR[s + 1] = vR.astype(jnp.bfloat16)
        else:
            part_ref[pl.ds(aR, BC), :] = vR
        rL.wait_recv()
        vL = part_ref[pl.ds(aL, BC), :] + bufBL[s].astype(jnp.float32)
        if s < NB - 2:
            sbBL[s + 1] = vL.astype(jnp.bfloat16)
        else:
            part_ref[pl.ds(aL, BC), :] = vL
        rR.wait_send()
        rL.wait_send()

    ownR = rbase + ((posB + 1) % NB) * BC
    ownL = lbase + ((posB - 1) % NB) * BC
    obf_ref[pl.ds(ownR, BC), :] = part_ref[pl.ds(ownR, BC), :].astype(
        jnp.bfloat16)
    obf_ref[pl.ds(ownL, BC), :] = part_ref[pl.ds(ownL, BC), :].astype(
        jnp.bfloat16)

    for s in range(NB - 1):
        sR = rbase + ((posB + 1 - s) % NB) * BC
        sL = lbase + ((posB - 1 + s) % NB) * BC
        rR = copy(obf_ref.at[pl.ds(sR, BC), :], obf_ref.at[pl.ds(sR, BC), :],
                  agB_sR.at[s], agB_rR.at[s], rtB)
        rL = copy(obf_ref.at[pl.ds(sL, BC), :], obf_ref.at[pl.ds(sL, BC), :],
                  agB_sL.at[s], agB_rL.at[s], ltB)
        rR.wait_recv()
        rL.wait_recv()
        rR.wait_send()
        rL.wait_send()

    def agA_send(d, q, s):
        if d == 0:
            rows = pl.ds(((posA + 1 - s) % NA) * AC + q * QH, QH)
            ss, rs, dev = agA_sR, agA_rR, rtA
        else:
            rows = pl.ds(((posA - 1 + s) % NA) * AC + H + q * QH, QH)
            ss, rs, dev = agA_sL, agA_rL, ltA
        return copy(obf_ref.at[rows, :], obf_ref.at[rows, :],
                    ss.at[2 * s + q], rs.at[2 * s + q], dev)

    gR = [[agA_send(0, q, 0) for q in (0, 1)]]
    gL = [[agA_send(1, q, 0) for q in (0, 1)]]
    for s in range(NA - 1):
        if s + 1 < NA - 1:
            gR.append([None, None])
            gL.append([None, None])
        for q in (0, 1):
            gR[s][q].wait_recv()
            if s + 1 < NA - 1:
                gR[s + 1][q] = agA_send(0, q, s + 1)
        for q in (0, 1):
            gL[s][q].wait_recv()
            if s + 1 < NA - 1:
                gL[s + 1][q] = agA_send(1, q, s + 1)
        for q in (0, 1):
            gR[s][q].wait_send()
            gL[s][q].wait_send()

    for b in range(B):
        out_ref[b, :, :] = obf_ref[pl.ds(b * S, S), :].astype(jnp.float32)


_TABLES = np.asarray([_POSA, _RTA, _LTA, _POSB, _RTB, _LTB], dtype=np.int32)


def kernel(x, k, Wp):
    scalars = jnp.asarray(_TABLES)

    sem7 = pltpu.SemaphoreType.DMA((2 * (NA - 1),))
    sem3 = pltpu.SemaphoreType.DMA((NB - 1,))
    out = pl.pallas_call(
        _body,
        out_shape=jax.ShapeDtypeStruct((B, S, OUT_N), jnp.float32),
        in_specs=[
            pl.BlockSpec(memory_space=pltpu.SMEM),
            pl.BlockSpec(memory_space=pltpu.VMEM),
            pl.BlockSpec(memory_space=pltpu.VMEM),
            pl.BlockSpec(memory_space=pltpu.VMEM),
        ],
        out_specs=pl.BlockSpec(memory_space=pltpu.VMEM),
        scratch_shapes=[
            pltpu.VMEM((B, S + 16, C), jnp.float32),
            pltpu.VMEM((ROWS, OUT_N), jnp.float32),
            pltpu.VMEM((ROWS, OUT_N), jnp.bfloat16),
            pltpu.VMEM((NA - 1, H, OUT_N), jnp.bfloat16),
            pltpu.VMEM((NA - 1, H, OUT_N), jnp.bfloat16),
            pltpu.VMEM((NB - 1, BC, OUT_N), jnp.bfloat16),
            pltpu.VMEM((NB - 1, BC, OUT_N), jnp.bfloat16),
            pltpu.VMEM((NA - 1, H, OUT_N), jnp.bfloat16),
            pltpu.VMEM((NA - 1, H, OUT_N), jnp.bfloat16),
            pltpu.VMEM((NB - 1, BC, OUT_N), jnp.bfloat16),
            pltpu.VMEM((NB - 1, BC, OUT_N), jnp.bfloat16),
            sem7, sem7, sem7, sem7,
            sem3, sem3, sem3, sem3,
            sem3, sem3, sem3, sem3,
            sem7, sem7, sem7, sem7,
        ],
        compiler_params=pltpu.CompilerParams(collective_id=0),
    )(scalars, x, k, Wp)
    return out


# device time: 88101 ns/iter; 3.4299x vs baseline; 1.0008x over previous
import jax
import jax.numpy as jnp
import numpy as np
from jax import lax
from jax.experimental import pallas as pl
from jax.experimental.pallas import tpu as pltpu

N = 32
B, S, C = 4, 1024, 512
OUT_N = 512
ROWS = B * S
TAPS = 4

NA = 8
NB = 4
AC = ROWS // NA
H = AC // 2
QH = H // 2
BC = H // NB


def _build_tables():
    fb = ([p // NB for p in range(N)],
          [((p // NB + 1) % NA) * NB + p % NB for p in range(N)],
          [((p // NB - 1) % NA) * NB + p % NB for p in range(N)],
          [p % NB for p in range(N)],
          [(p // NB) * NB + (p + 1) % NB for p in range(N)],
          [(p // NB) * NB + (p - 1) % NB for p in range(N)])
    try:
        import distributed_mesh_v7x as dm
        mesh = dm.get_mesh("i", world_size=N)
        devs = list(mesh.devices.flat)
        coords = [tuple(d.coords) for d in devs]
        if len(set(coords)) != N or any(len(c) != 3 for c in coords):
            return fb
        axes = [sorted({c[i] for c in coords}) for i in range(3)]
        sizes = [len(a) for a in axes]
        if sorted(sizes) != [2, 4, 4]:
            return fb
        a2 = sizes.index(2)
        a4 = [i for i in range(3) if i != a2]
        us = axes[a4[0]]
        _v = axes[a4[1]]
        vs = [_v[0], _v[1], _v[3], _v[2]]
        lo, hi = axes[a2]
        cyc = [(lo, u) for u in us] + [(hi, u) for u in reversed(us)]
        posA_of = {xu: i for i, xu in enumerate(cyc)}
        log_of = {c: p for p, c in enumerate(coords)}

        def at(c, i2, iu, iv):
            t = [0, 0, 0]
            t[a2], t[a4[0]], t[a4[1]] = i2, iu, iv
            return tuple(t)

        posA = [0] * N
        rtA = [0] * N
        ltA = [0] * N
        posB = [0] * N
        rtB = [0] * N
        ltB = [0] * N
        for p, c in enumerate(coords):
            i2, iu, iv = c[a2], c[a4[0]], c[a4[1]]
            pa = posA_of[(i2, iu)]
            pb = vs.index(iv)
            posA[p] = pa
            posB[p] = pb
            nxt = cyc[(pa + 1) % NA]
            prv = cyc[(pa - 1) % NA]
            rtA[p] = log_of[at(c, nxt[0], nxt[1], iv)]
            ltA[p] = log_of[at(c, prv[0], prv[1], iv)]
            rtB[p] = log_of[at(c, i2, iu, vs[(pb + 1) % NB])]
            ltB[p] = log_of[at(c, i2, iu, vs[(pb - 1) % NB])]
        return posA, rtA, ltA, posB, rtB, ltB
    except Exception:
        return fb


_POSA, _RTA, _LTA, _POSB, _RTB, _LTB = _build_tables()


def _body(scal_ref, x_ref, k_ref, wp_ref, out_ref,
          pad_ref, part_ref, obf_ref, bufAR, bufAL, bufBR, bufBL,
          sbAR, sbAL, sbBR, sbBL,
          rsA_sR, rsA_rR, rsA_sL, rsA_rL,
          rsB_sR, rsB_rR, rsB_sL, rsB_rL,
          agB_sR, agB_rR, agB_sL, agB_rL,
          agA_sR, agA_rR, agA_sL, agA_rL,
          stage_ref, cp_sems):
    my = lax.axis_index("i")
    posA = scal_ref[0, my]
    rtA = scal_ref[1, my]
    ltA = scal_ref[2, my]
    posB = scal_ref[3, my]
    rtB = scal_ref[4, my]
    ltB = scal_ref[5, my]

    barrier_sem = pltpu.get_barrier_semaphore()
    for nbr in (rtA, ltA, rtB, ltB):
        pl.semaphore_signal(barrier_sem, inc=1, device_id=(nbr,),
                            device_id_type=pl.DeviceIdType.MESH)

    kv = k_ref[:, :]
    wpv = wp_ref[:, :]
    for b in range(B):
        pad_ref[b, 0:TAPS - 1, :] = jnp.zeros((TAPS - 1, C), jnp.float32)
        pad_ref[b, TAPS - 1:TAPS - 1 + S, :] = x_ref[b]

    def compute_half(c, half):
        b = c // 2
        rl = (c % 2) * AC + half * H
        w = pad_ref[b, pl.ds(rl, H + 8), :]
        acc = w[0:H, :] * kv[0:1, :]
        for t in range(1, TAPS):
            acc = acc + w[t:t + H, :] * kv[t:t + 1, :]
        a = acc * (1.0 / (1.0 + jnp.exp(-acc)))
        part_ref[pl.ds(c * AC + half * H, H), :] = jnp.dot(
            a, wpv, preferred_element_type=jnp.float32)

    compute_half(posA % NA, 0)
    compute_half(posA % NA, 1)

    pl.semaphore_wait(barrier_sem, 4)

    def copy(src, dst, ssem, rsem, dev):
        r = pltpu.make_async_remote_copy(
            src_ref=src, dst_ref=dst, send_sem=ssem, recv_sem=rsem,
            device_id=dev, device_id_type=pl.DeviceIdType.LOGICAL)
        r.start()
        return r

    NSLOT = 8
    cps = []

    def emit(row, n):
        slot = len(cps) % NSLOT
        if len(cps) >= NSLOT:
            cps[len(cps) - NSLOT].wait()
        stage_ref[slot, 0:n, :] = obf_ref[pl.ds(row, n), :].astype(
            jnp.float32)
        cp = pltpu.make_async_copy(
            stage_ref.at[slot, pl.ds(0, n), :],
            out_ref.at[row // S, pl.ds(row % S, n), :],
            cp_sems.at[slot])
        cp.start()
        cps.append(cp)

    sbAR[0] = part_ref[pl.ds((posA % NA) * AC, H), :].astype(jnp.bfloat16)
    sbAL[0] = part_ref[pl.ds((posA % NA) * AC + H, H), :].astype(jnp.bfloat16)

    def rsA_send(d, q, s):
        sb, buf, ss, rs, dev = (
            (sbAR, bufAR, rsA_sR, rsA_rR, rtA) if d == 0 else
            (sbAL, bufAL, rsA_sL, rsA_rL, ltA))
        return copy(sb.at[s, pl.ds(q * QH, QH), :],
                    buf.at[s, pl.ds(q * QH, QH), :],
                    ss.at[2 * s + q], rs.at[2 * s + q], dev)

    dR = [[rsA_send(0, q, 0) for q in (0, 1)]]
    dL = [[rsA_send(1, q, 0) for q in (0, 1)]]
    compute_half((posA - 1) % NA, 0)
    compute_half((posA + 1) % NA, 1)
    for s in range(NA - 1):
        crR = ((posA - s - 1) % NA) * AC
        crL = ((posA + s + 1) % NA) * AC + H
        if s + 1 < NA - 1:
            dR.append([None, None])
            dL.append([None, None])
        for q in (0, 1):
            dR[s][q].wait_recv()
            rows = pl.ds(crR + q * QH, QH)
            v = part_ref[rows, :] + bufAR[s, q * QH:(q + 1) * QH, :].astype(
                jnp.float32)
            if s < NA - 2:
                sbAR[s + 1, q * QH:(q + 1) * QH, :] = v.astype(jnp.bfloat16)
                dR[s + 1][q] = rsA_send(0, q, s + 1)
            else:
                part_ref[rows, :] = v
        for q in (0, 1):
            dL[s][q].wait_recv()
            rows = pl.ds(crL + q * QH, QH)
            v = part_ref[rows, :] + bufAL[s, q * QH:(q + 1) * QH, :].astype(
                jnp.float32)
            if s < NA - 2:
                sbAL[s + 1, q * QH:(q + 1) * QH, :] = v.astype(jnp.bfloat16)
                dL[s + 1][q] = rsA_send(1, q, s + 1)
            else:
                part_ref[rows, :] = v
        if s < NA - 2:
            compute_half((posA - s - 2) % NA, 0)
            compute_half((posA + s + 2) % NA, 1)
        for q in (0, 1):
            dR[s][q].wait_send()
            dL[s][q].wait_send()

    rbase = ((posA + 1) % NA) * AC
    lbase = ((posA - 1) % NA) * AC + H

    sbBR[0] = part_ref[pl.ds(rbase + (posB % NB) * BC, BC), :].astype(
        jnp.bfloat16)
    sbBL[0] = part_ref[pl.ds(lbase + (posB % NB) * BC, BC), :].astype(
        jnp.bfloat16)
    for s in range(NB - 1):
        aR = rbase + ((posB - s - 1) % NB) * BC
        aL = lbase + ((posB + s + 1) % NB) * BC
        rR = copy(sbBR.at[s], bufBR.at[s],
                  rsB_sR.at[s], rsB_rR.at[s], rtB)
        rL = copy(sbBL.at[s], bufBL.at[s],
                  rsB_sL.at[s], rsB_rL.at[s], ltB)
        rR.wait_recv()
        vR = part_ref[pl.ds(aR, BC), :] + bufBR[s].astype(jnp.float32)
        if s < NB - 2:
            sbBR[s + 1] = vR.astype(jnp.bfloat16)
        else:
            part_ref[pl.ds(aR, BC), :] = vR
        rL.wait_recv()
        vL = part_ref[pl.ds(aL, BC), :] + bufBL[s].astype(jnp.float32)
        if s < NB - 2:
            sbBL[s + 1] = vL.astype(jnp.bfloat16)
        else:
            part_ref[pl.ds(aL, BC), :] = vL
        rR.wait_send()
        rL.wait_send()

    ownR = rbase + ((posB + 1) % NB) * BC
    ownL = lbase + ((posB - 1) % NB) * BC
    obf_ref[pl.ds(ownR, BC), :] = part_ref[pl.ds(ownR, BC), :].astype(
        jnp.bfloat16)
    obf_ref[pl.ds(ownL, BC), :] = part_ref[pl.ds(ownL, BC), :].astype(
        jnp.bfloat16)

    for s in range(NB - 1):
        sR = rbase + ((posB + 1 - s) % NB) * BC
        sL = lbase + ((posB - 1 + s) % NB) * BC
        rR = copy(obf_ref.at[pl.ds(sR, BC), :], obf_ref.at[pl.ds(sR, BC), :],
                  agB_sR.at[s], agB_rR.at[s], rtB)
        rL = copy(obf_ref.at[pl.ds(sL, BC), :], obf_ref.at[pl.ds(sL, BC), :],
                  agB_sL.at[s], agB_rL.at[s], ltB)
        rR.wait_recv()
        rL.wait_recv()
        rR.wait_send()
        rL.wait_send()

    emit(rbase, QH)
    emit(rbase + QH, QH)
    emit(lbase, QH)
    emit(lbase + QH, QH)

    def agA_send(d, q, s):
        if d == 0:
            rows = pl.ds(((posA + 1 - s) % NA) * AC + q * QH, QH)
            ss, rs, dev = agA_sR, agA_rR, rtA
        else:
            rows = pl.ds(((posA - 1 + s) % NA) * AC + H + q * QH, QH)
            ss, rs, dev = agA_sL, agA_rL, ltA
        return copy(obf_ref.at[rows, :], obf_ref.at[rows, :],
                    ss.at[2 * s + q], rs.at[2 * s + q], dev)

    gR = [[agA_send(0, q, 0) for q in (0, 1)]]
    gL = [[agA_send(1, q, 0) for q in (0, 1)]]
    for s in range(NA - 1):
        if s + 1 < NA - 1:
            gR.append([None, None])
            gL.append([None, None])
        for q in (0, 1):
            gR[s][q].wait_recv()
            if s + 1 < NA - 1:
                gR[s + 1][q] = agA_send(0, q, s + 1)
        for q in (0, 1):
            gL[s][q].wait_recv()
            if s + 1 < NA - 1:
                gL[s + 1][q] = agA_send(1, q, s + 1)
        for q in (0, 1):
            emit(((posA - s) % NA) * AC + q * QH, QH)
            emit(((posA + s) % NA) * AC + H + q * QH, QH)
        for q in (0, 1):
            gR[s][q].wait_send()
            gL[s][q].wait_send()

    for cp in cps[-NSLOT:]:
        cp.wait()


_TABLES = np.asarray([_POSA, _RTA, _LTA, _POSB, _RTB, _LTB], dtype=np.int32)


def kernel(x, k, Wp):
    scalars = jnp.asarray(_TABLES)

    sem7 = pltpu.SemaphoreType.DMA((2 * (NA - 1),))
    sem3 = pltpu.SemaphoreType.DMA((NB - 1,))
    out = pl.pallas_call(
        _body,
        out_shape=jax.ShapeDtypeStruct((B, S, OUT_N), jnp.float32),
        in_specs=[
            pl.BlockSpec(memory_space=pltpu.SMEM),
            pl.BlockSpec(memory_space=pltpu.VMEM),
            pl.BlockSpec(memory_space=pltpu.VMEM),
            pl.BlockSpec(memory_space=pltpu.VMEM),
        ],
        out_specs=pl.BlockSpec(memory_space=pl.ANY),
        scratch_shapes=[
            pltpu.VMEM((B, S + 16, C), jnp.float32),
            pltpu.VMEM((ROWS, OUT_N), jnp.float32),
            pltpu.VMEM((ROWS, OUT_N), jnp.bfloat16),
            pltpu.VMEM((NA - 1, H, OUT_N), jnp.bfloat16),
            pltpu.VMEM((NA - 1, H, OUT_N), jnp.bfloat16),
            pltpu.VMEM((NB - 1, BC, OUT_N), jnp.bfloat16),
            pltpu.VMEM((NB - 1, BC, OUT_N), jnp.bfloat16),
            pltpu.VMEM((NA - 1, H, OUT_N), jnp.bfloat16),
            pltpu.VMEM((NA - 1, H, OUT_N), jnp.bfloat16),
            pltpu.VMEM((NB - 1, BC, OUT_N), jnp.bfloat16),
            pltpu.VMEM((NB - 1, BC, OUT_N), jnp.bfloat16),
            sem7, sem7, sem7, sem7,
            sem3, sem3, sem3, sem3,
            sem3, sem3, sem3, sem3,
            sem7, sem7, sem7, sem7,
            pltpu.VMEM((8, QH, OUT_N), jnp.float32),
            pltpu.SemaphoreType.DMA((8,)),
        ],
        compiler_params=pltpu.CompilerParams(collective_id=0),
    )(scalars, x, k, Wp)
    return out
